# baseline (speedup 1.0000x reference)
import functools

import jax
import jax.numpy as jnp
from jax import lax
from jax.experimental import pallas as pl
from jax.experimental.pallas import tpu as pltpu
from jax.experimental.pallas import tpu_sc as plsc

N_NODES = 10000
NP = 10112
N_EDGES = 320000
D = 128
DH = 64
D_OUT = 64

NC = 2
NS = 16
NW = NC * NS

CHUNK = 128
CPW = 79
E_PAD = NW * CPW * CHUNK
JUNK = N_NODES

STRIPE = NP // NS
ZROWS = 8

DEG_LEN = 10240
DSTRIPE = DEG_LEN // NS

BLK = 1264
_GRID = NP // BLK


def _deg_body(dst_hbm, out_hbm, idx_d, ones_v, zstripe, deg_sh):
    c = lax.axis_index("c")
    s = lax.axis_index("s")
    wid = c * NS + s

    def fill_ones(i, _):
        ones_v[pl.ds(i * 16, 16)] = jnp.ones((16,), jnp.float32)
        return 0

    lax.fori_loop(0, CHUNK // 16, fill_ones, 0)

    def fill_z(i, _):
        zstripe[pl.ds(i * 16, 16)] = jnp.zeros((16,), jnp.float32)
        return 0

    lax.fori_loop(0, DSTRIPE // 16, fill_z, 0)

    pltpu.sync_copy(dst_hbm.at[wid], idx_d)
    pltpu.sync_copy(zstripe, deg_sh.at[pl.ds(s * DSTRIPE, DSTRIPE)])
    plsc.subcore_barrier()

    def body(j, _):
        pltpu.sync_copy(ones_v, deg_sh.at[idx_d.at[j]], add=True)
        return 0

    lax.fori_loop(0, CPW, body, 0)
    plsc.subcore_barrier()
    pltpu.sync_copy(
        deg_sh.at[pl.ds(s * DSTRIPE, DSTRIPE)],
        out_hbm.at[c].at[pl.ds(s * DSTRIPE, DSTRIPE)],
    )


_deg_kernel = functools.partial(
    pl.kernel,
    out_type=jax.ShapeDtypeStruct((NC, DEG_LEN), jnp.float32),
    mesh=plsc.VectorSubcoreMesh(core_axis_name="c", subcore_axis_name="s"),
    scratch_types=[
        pltpu.VMEM((CPW, CHUNK), jnp.int32),
        pltpu.VMEM((CHUNK,), jnp.float32),
        pltpu.VMEM((DSTRIPE,), jnp.float32),
        pltpu.VMEM_SHARED((DEG_LEN,), jnp.float32),
    ],
)(_deg_body)


G = 4
NB = 8


def _agg_body(lo_hbm, hi_hbm, src_hbm, dst_hbm, out_hbm, idx_s, idx_d, rows, zbuf, acc_sh, gsem, ssem):
    c = lax.axis_index("c")
    s = lax.axis_index("s")
    wid = c * NS + s

    def fill_z(i, _):
        zbuf[i // 4, pl.ds((i % 4) * 16, 16)] = jnp.zeros((16,), jnp.float32)
        return 0

    lax.fori_loop(0, ZROWS * 4, fill_z, 0)

    pltpu.sync_copy(src_hbm.at[wid], idx_s)
    pltpu.sync_copy(dst_hbm.at[wid], idx_d)

    for half, hs_hbm in ((0, lo_hbm), (1, hi_hbm)):
        for k in range(STRIPE // ZROWS):
            pltpu.sync_copy(zbuf, acc_sh.at[pl.ds(s * STRIPE + k * ZROWS, ZROWS)])
        plsc.subcore_barrier()

        def wait_gather():
            pltpu.make_async_copy(hs_hbm.at[idx_s.at[0]], rows.at[0], gsem).wait()

        def wait_scatter():
            pltpu.make_async_copy(rows.at[0], acc_sh.at[idx_d.at[0]], ssem).wait()

        for j in range(G):
            pltpu.async_copy(hs_hbm.at[idx_s.at[j]], rows.at[j % NB], gsem)

        def body(j, _):
            wait_gather()
            @pl.when(j >= NB - G)
            def _():
                wait_scatter()

            pltpu.async_copy(hs_hbm.at[idx_s.at[j + G]], rows.at[(j + G) % NB], gsem)
            pltpu.async_copy(rows.at[j % NB], acc_sh.at[idx_d.at[j]], ssem, add=True)
            return 0

        lax.fori_loop(0, CPW - G, body, 0)
        for j in range(CPW - G, CPW):
            wait_gather()
            wait_scatter()
            pltpu.async_copy(rows.at[j % NB], acc_sh.at[idx_d.at[j]], ssem, add=True)
        for _ in range(G):
            wait_scatter()

        plsc.subcore_barrier()
        pltpu.sync_copy(
            acc_sh.at[pl.ds(s * STRIPE, STRIPE)],
            out_hbm.at[c].at[half].at[pl.ds(s * STRIPE, STRIPE)],
        )
        if half == 0:
            plsc.subcore_barrier()


_agg_kernel = functools.partial(
    pl.kernel,
    out_type=jax.ShapeDtypeStruct((NC, 2, NP, DH), jnp.float32),
    mesh=plsc.VectorSubcoreMesh(core_axis_name="c", subcore_axis_name="s"),
    scratch_types=[
        pltpu.VMEM((CPW, CHUNK), jnp.int32),
        pltpu.VMEM((CPW, CHUNK), jnp.int32),
        pltpu.VMEM((NB, CHUNK, DH), jnp.float32),
        pltpu.VMEM((ZROWS, DH), jnp.float32),
        pltpu.VMEM_SHARED((NP, DH), jnp.float32),
        pltpu.SemaphoreType.DMA,
        pltpu.SemaphoreType.DMA,
    ],
    compiler_params=pltpu.CompilerParams(use_tc_tiling_on_sc=False),
)(_agg_body)


def _dinv(deg_ref):
    return lax.rsqrt(jnp.sum(deg_ref[...], axis=1) + 1.0)


def _split_store(o_lo, o_hi, v):
    o_lo[...] = v[:, :DH]
    o_hi[...] = v[:, DH:]


def _prep_body(x_ref, w_ref, deg_ref, o_lo, o_hi):
    dinv = _dinv(deg_ref)
    h = jnp.dot(x_ref[...], w_ref[...], preferred_element_type=jnp.float32)
    _split_store(o_lo, o_hi, h * dinv[:, None])


def _mid_body(agg_ref, lo_ref, hi_ref, deg_ref, b_ref, w_ref, o_lo, o_hi):
    dinv = _dinv(deg_ref)
    hs = jnp.concatenate([lo_ref[...], hi_ref[...]], axis=1)
    lo = agg_ref[0, 0] + agg_ref[1, 0]
    hi = agg_ref[0, 1] + agg_ref[1, 1]
    p = (jnp.concatenate([lo, hi], axis=1) + hs) * dinv[:, None] + b_ref[...]
    h = jnp.maximum(p, 0.0)
    out = jnp.dot(h, w_ref[...], preferred_element_type=jnp.float32) * dinv[:, None]
    _split_store(o_lo, o_hi, out)


def _final_body(agg_ref, lo_ref, hi_ref, deg_ref, b_ref, wc_ref, bc_ref, o_ref, acc):
    i = pl.program_id(0)
    dinv = _dinv(deg_ref)
    hs = jnp.concatenate([lo_ref[...], hi_ref[...]], axis=1)
    lo = agg_ref[0, 0] + agg_ref[1, 0]
    hi = agg_ref[0, 1] + agg_ref[1, 1]
    p = (jnp.concatenate([lo, hi], axis=1) + hs) * dinv[:, None] + b_ref[...]
    h = jnp.maximum(p, 0.0)
    row = lax.broadcasted_iota(jnp.int32, (BLK, 1), 0) + i * BLK
    h = jnp.where(row < N_NODES, h, 0.0)
    part = jnp.sum(h, axis=0, keepdims=True)

    @pl.when(i == 0)
    def _():
        acc[...] = part

    @pl.when(i > 0)
    def _():
        acc[...] = acc[...] + part

    @pl.when(i == pl.num_programs(0) - 1)
    def _():
        pooled = acc[...] * (1.0 / N_NODES)
        o_ref[...] = (
            jnp.dot(pooled, wc_ref[...], preferred_element_type=jnp.float32)
            + bc_ref[...]
        )


_row_spec = pl.BlockSpec((BLK, D), lambda i: (i, 0))
_half_spec = pl.BlockSpec((BLK, DH), lambda i: (i, 0))
_w_spec = pl.BlockSpec((D, D), lambda i: (0, 0))
_deg_spec = pl.BlockSpec((BLK, NC), lambda i: (i, 0))
_agg_spec = pl.BlockSpec((NC, 2, BLK, DH), lambda i: (0, 0, i, 0))
_b_spec = pl.BlockSpec((D,), lambda i: (0,))

_halves_t = (
    jax.ShapeDtypeStruct((NP, DH), jnp.float32),
    jax.ShapeDtypeStruct((NP, DH), jnp.float32),
)

_prep = pl.pallas_call(
    _prep_body,
    grid=(_GRID,),
    in_specs=[_row_spec, _w_spec, _deg_spec],
    out_specs=(_half_spec, _half_spec),
    out_shape=_halves_t,
)

_mid = pl.pallas_call(
    _mid_body,
    grid=(_GRID,),
    in_specs=[_agg_spec, _half_spec, _half_spec, _deg_spec, _b_spec, _w_spec],
    out_specs=(_half_spec, _half_spec),
    out_shape=_halves_t,
)

_final = pl.pallas_call(
    _final_body,
    grid=(_GRID,),
    in_specs=[
        _agg_spec,
        _half_spec,
        _half_spec,
        _deg_spec,
        _b_spec,
        pl.BlockSpec((D, D_OUT), lambda i: (0, 0)),
        pl.BlockSpec((D_OUT,), lambda i: (0,)),
    ],
    out_specs=pl.BlockSpec((1, D_OUT), lambda i: (0, 0)),
    out_shape=jax.ShapeDtypeStruct((1, D_OUT), jnp.float32),
    scratch_shapes=[pltpu.VMEM((1, D), jnp.float32)],
)


def kernel(x, edge_index, W1, b1, W2, b2, Wc, bc):
    pad = E_PAD - N_EDGES
    src = jnp.concatenate(
        [edge_index[0].astype(jnp.int32), jnp.zeros((pad,), jnp.int32)]
    ).reshape(NW, CPW, CHUNK)
    dst = jnp.concatenate(
        [edge_index[1].astype(jnp.int32), jnp.full((pad,), JUNK, jnp.int32)]
    ).reshape(NW, CPW, CHUNK)

    xp = jnp.zeros((NP, D), jnp.float32).at[:N_NODES].set(x)

    degp = _deg_kernel(dst)
    deg = jnp.zeros((NP, NC), jnp.float32).at[:N_NODES].set(degp[:, :N_NODES].T)

    hs1_lo, hs1_hi = _prep(xp, W1, deg)
    agg1 = _agg_kernel(hs1_lo, hs1_hi, src, dst)
    hs2_lo, hs2_hi = _mid(agg1, hs1_lo, hs1_hi, deg, b1, W2)
    agg2 = _agg_kernel(hs2_lo, hs2_hi, src, dst)
    out = _final(agg2, hs2_lo, hs2_hi, deg, b2, Wc, bc)
    return out.reshape(D_OUT)


# speedup vs baseline: 1.0190x; 1.0190x over previous
import functools

import jax
import jax.numpy as jnp
from jax import lax
from jax.experimental import pallas as pl
from jax.experimental.pallas import tpu as pltpu
from jax.experimental.pallas import tpu_sc as plsc

N_NODES = 10000
NP = 10112
N_EDGES = 320000
D = 128
DH = 64
D_OUT = 64

NC = 2
NS = 16
NW = NC * NS

CHUNK = 128
CPW = 79
E_PAD = NW * CPW * CHUNK
JUNK = N_NODES

STRIPE = NP // NS
ZROWS = 158

DEG_LEN = 10240
DSTRIPE = DEG_LEN // NS

BLK = 1264
_GRID = NP // BLK


def _deg_body(dst_hbm, out_hbm, idx_d, ones_v, zstripe, deg_sh):
    c = lax.axis_index("c")
    s = lax.axis_index("s")
    wid = c * NS + s

    def fill_ones(i, _):
        ones_v[pl.ds(i * 16, 16)] = jnp.ones((16,), jnp.float32)
        return 0

    lax.fori_loop(0, CHUNK // 16, fill_ones, 0)

    def fill_z(i, _):
        zstripe[pl.ds(i * 16, 16)] = jnp.zeros((16,), jnp.float32)
        return 0

    lax.fori_loop(0, DSTRIPE // 16, fill_z, 0)

    pltpu.sync_copy(dst_hbm.at[wid], idx_d)
    pltpu.sync_copy(zstripe, deg_sh.at[pl.ds(s * DSTRIPE, DSTRIPE)])
    plsc.subcore_barrier()

    def body(j, _):
        pltpu.sync_copy(ones_v, deg_sh.at[idx_d.at[j]], add=True)
        return 0

    lax.fori_loop(0, CPW, body, 0)
    plsc.subcore_barrier()
    pltpu.sync_copy(
        deg_sh.at[pl.ds(s * DSTRIPE, DSTRIPE)],
        out_hbm.at[c].at[pl.ds(s * DSTRIPE, DSTRIPE)],
    )


_deg_kernel = functools.partial(
    pl.kernel,
    out_type=jax.ShapeDtypeStruct((NC, DEG_LEN), jnp.float32),
    mesh=plsc.VectorSubcoreMesh(core_axis_name="c", subcore_axis_name="s"),
    scratch_types=[
        pltpu.VMEM((CPW, CHUNK), jnp.int32),
        pltpu.VMEM((CHUNK,), jnp.float32),
        pltpu.VMEM((DSTRIPE,), jnp.float32),
        pltpu.VMEM_SHARED((DEG_LEN,), jnp.float32),
    ],
)(_deg_body)


G = 4
NB = 7


def _agg_body(lo_hbm, hi_hbm, src_hbm, dst_hbm, out_hbm, idx_s, idx_d, rows, zbuf, acc_sh, gsem, ssem):
    c = lax.axis_index("c")
    s = lax.axis_index("s")
    wid = c * NS + s

    def fill_z(i, _):
        zbuf[i // 4, pl.ds((i % 4) * 16, 16)] = jnp.zeros((16,), jnp.float32)
        return 0

    lax.fori_loop(0, ZROWS * 4, fill_z, 0)

    pltpu.sync_copy(src_hbm.at[wid], idx_s)
    pltpu.sync_copy(dst_hbm.at[wid], idx_d)

    for half, hs_hbm in ((0, lo_hbm), (1, hi_hbm)):
        for k in range(STRIPE // ZROWS):
            pltpu.sync_copy(zbuf, acc_sh.at[pl.ds(s * STRIPE + k * ZROWS, ZROWS)])
        plsc.subcore_barrier()

        def wait_gather():
            pltpu.make_async_copy(hs_hbm.at[idx_s.at[0]], rows.at[0], gsem).wait()

        def wait_scatter():
            pltpu.make_async_copy(rows.at[0], acc_sh.at[idx_d.at[0]], ssem).wait()

        for j in range(G):
            pltpu.async_copy(hs_hbm.at[idx_s.at[j]], rows.at[j % NB], gsem)

        def body(j, _):
            wait_gather()
            @pl.when(j >= NB - G)
            def _():
                wait_scatter()

            pltpu.async_copy(hs_hbm.at[idx_s.at[j + G]], rows.at[(j + G) % NB], gsem)
            pltpu.async_copy(rows.at[j % NB], acc_sh.at[idx_d.at[j]], ssem, add=True)
            return 0

        lax.fori_loop(0, CPW - G, body, 0)
        for j in range(CPW - G, CPW):
            wait_gather()
            wait_scatter()
            pltpu.async_copy(rows.at[j % NB], acc_sh.at[idx_d.at[j]], ssem, add=True)
        for _ in range(NB - G):
            wait_scatter()

        plsc.subcore_barrier()
        pltpu.sync_copy(
            acc_sh.at[pl.ds(s * STRIPE, STRIPE)],
            out_hbm.at[c].at[half].at[pl.ds(s * STRIPE, STRIPE)],
        )
        if half == 0:
            plsc.subcore_barrier()


_agg_kernel = functools.partial(
    pl.kernel,
    out_type=jax.ShapeDtypeStruct((NC, 2, NP, DH), jnp.float32),
    mesh=plsc.VectorSubcoreMesh(core_axis_name="c", subcore_axis_name="s"),
    scratch_types=[
        pltpu.VMEM((CPW, CHUNK), jnp.int32),
        pltpu.VMEM((CPW, CHUNK), jnp.int32),
        pltpu.VMEM((NB, CHUNK, DH), jnp.float32),
        pltpu.VMEM((ZROWS, DH), jnp.float32),
        pltpu.VMEM_SHARED((NP, DH), jnp.float32),
        pltpu.SemaphoreType.DMA,
        pltpu.SemaphoreType.DMA,
    ],
    compiler_params=pltpu.CompilerParams(use_tc_tiling_on_sc=False),
)(_agg_body)


def _dinv(deg_ref):
    return lax.rsqrt(jnp.sum(deg_ref[...], axis=1) + 1.0)


def _split_store(o_lo, o_hi, v):
    o_lo[...] = v[:, :DH]
    o_hi[...] = v[:, DH:]


def _prep_body(x_ref, w_ref, deg_ref, o_lo, o_hi):
    dinv = _dinv(deg_ref)
    h = jnp.dot(x_ref[...], w_ref[...], preferred_element_type=jnp.float32)
    _split_store(o_lo, o_hi, h * dinv[:, None])


def _mid_body(agg_ref, lo_ref, hi_ref, deg_ref, b_ref, w_ref, o_lo, o_hi):
    dinv = _dinv(deg_ref)
    hs = jnp.concatenate([lo_ref[...], hi_ref[...]], axis=1)
    lo = agg_ref[0, 0] + agg_ref[1, 0]
    hi = agg_ref[0, 1] + agg_ref[1, 1]
    p = (jnp.concatenate([lo, hi], axis=1) + hs) * dinv[:, None] + b_ref[...]
    h = jnp.maximum(p, 0.0)
    out = jnp.dot(h, w_ref[...], preferred_element_type=jnp.float32) * dinv[:, None]
    _split_store(o_lo, o_hi, out)


def _final_body(agg_ref, lo_ref, hi_ref, deg_ref, b_ref, wc_ref, bc_ref, o_ref, acc):
    i = pl.program_id(0)
    dinv = _dinv(deg_ref)
    hs = jnp.concatenate([lo_ref[...], hi_ref[...]], axis=1)
    lo = agg_ref[0, 0] + agg_ref[1, 0]
    hi = agg_ref[0, 1] + agg_ref[1, 1]
    p = (jnp.concatenate([lo, hi], axis=1) + hs) * dinv[:, None] + b_ref[...]
    h = jnp.maximum(p, 0.0)
    row = lax.broadcasted_iota(jnp.int32, (BLK, 1), 0) + i * BLK
    h = jnp.where(row < N_NODES, h, 0.0)
    part = jnp.sum(h, axis=0, keepdims=True)

    @pl.when(i == 0)
    def _():
        acc[...] = part

    @pl.when(i > 0)
    def _():
        acc[...] = acc[...] + part

    @pl.when(i == pl.num_programs(0) - 1)
    def _():
        pooled = acc[...] * (1.0 / N_NODES)
        o_ref[...] = (
            jnp.dot(pooled, wc_ref[...], preferred_element_type=jnp.float32)
            + bc_ref[...]
        )


_row_spec = pl.BlockSpec((BLK, D), lambda i: (i, 0))
_half_spec = pl.BlockSpec((BLK, DH), lambda i: (i, 0))
_w_spec = pl.BlockSpec((D, D), lambda i: (0, 0))
_deg_spec = pl.BlockSpec((BLK, NC), lambda i: (i, 0))
_agg_spec = pl.BlockSpec((NC, 2, BLK, DH), lambda i: (0, 0, i, 0))
_b_spec = pl.BlockSpec((D,), lambda i: (0,))

_halves_t = (
    jax.ShapeDtypeStruct((NP, DH), jnp.float32),
    jax.ShapeDtypeStruct((NP, DH), jnp.float32),
)

_prep = pl.pallas_call(
    _prep_body,
    grid=(_GRID,),
    in_specs=[_row_spec, _w_spec, _deg_spec],
    out_specs=(_half_spec, _half_spec),
    out_shape=_halves_t,
)

_mid = pl.pallas_call(
    _mid_body,
    grid=(_GRID,),
    in_specs=[_agg_spec, _half_spec, _half_spec, _deg_spec, _b_spec, _w_spec],
    out_specs=(_half_spec, _half_spec),
    out_shape=_halves_t,
)

_final = pl.pallas_call(
    _final_body,
    grid=(_GRID,),
    in_specs=[
        _agg_spec,
        _half_spec,
        _half_spec,
        _deg_spec,
        _b_spec,
        pl.BlockSpec((D, D_OUT), lambda i: (0, 0)),
        pl.BlockSpec((D_OUT,), lambda i: (0,)),
    ],
    out_specs=pl.BlockSpec((1, D_OUT), lambda i: (0, 0)),
    out_shape=jax.ShapeDtypeStruct((1, D_OUT), jnp.float32),
    scratch_shapes=[pltpu.VMEM((1, D), jnp.float32)],
)


def kernel(x, edge_index, W1, b1, W2, b2, Wc, bc):
    pad = E_PAD - N_EDGES
    src = jnp.concatenate(
        [edge_index[0].astype(jnp.int32), jnp.zeros((pad,), jnp.int32)]
    ).reshape(NW, CPW, CHUNK)
    dst = jnp.concatenate(
        [edge_index[1].astype(jnp.int32), jnp.full((pad,), JUNK, jnp.int32)]
    ).reshape(NW, CPW, CHUNK)

    xp = jnp.zeros((NP, D), jnp.float32).at[:N_NODES].set(x)

    degp = _deg_kernel(dst)
    deg = jnp.zeros((NP, NC), jnp.float32).at[:N_NODES].set(degp[:, :N_NODES].T)

    hs1_lo, hs1_hi = _prep(xp, W1, deg)
    agg1 = _agg_kernel(hs1_lo, hs1_hi, src, dst)
    hs2_lo, hs2_hi = _mid(agg1, hs1_lo, hs1_hi, deg, b1, W2)
    agg2 = _agg_kernel(hs2_lo, hs2_hi, src, dst)
    out = _final(agg2, hs2_lo, hs2_hi, deg, b2, Wc, bc)
    return out.reshape(D_OUT)


# speedup vs baseline: 1.9346x; 1.8985x over previous
import functools

import jax
import jax.numpy as jnp
from jax import lax
from jax.experimental import pallas as pl
from jax.experimental.pallas import tpu as pltpu
from jax.experimental.pallas import tpu_sc as plsc

N_NODES = 10000
NP = 10112
N_EDGES = 320000
D = 128
DH = 64
D_OUT = 64

NC = 2
NS = 16
NW = NC * NS

CHUNK = 100
CPW = 100

STRIPE = NP // NS
ZROWS = 158

DEG_LEN = 10240
DSTRIPE = DEG_LEN // NS

BLK = 1264
_GRID = NP // BLK


def _deg_body(dst_hbm, out_hbm, idx_d, ones_v, zstripe, deg_sh):
    c = lax.axis_index("c")
    s = lax.axis_index("s")
    wid = c * NS + s

    def fill_ones(i, _):
        ones_v[pl.ds(i * 16, 16)] = jnp.ones((16,), jnp.float32)
        return 0

    lax.fori_loop(0, CHUNK // 16, fill_ones, 0)

    def fill_z(i, _):
        zstripe[pl.ds(i * 16, 16)] = jnp.zeros((16,), jnp.float32)
        return 0

    lax.fori_loop(0, DSTRIPE // 16, fill_z, 0)

    pltpu.sync_copy(dst_hbm.at[wid], idx_d)
    pltpu.sync_copy(zstripe, deg_sh.at[pl.ds(s * DSTRIPE, DSTRIPE)])
    plsc.subcore_barrier()

    def body(j, _):
        pltpu.sync_copy(ones_v, deg_sh.at[idx_d.at[j]], add=True)
        return 0

    lax.fori_loop(0, CPW, body, 0)
    plsc.subcore_barrier()
    pltpu.sync_copy(
        deg_sh.at[pl.ds(s * DSTRIPE, DSTRIPE)],
        out_hbm.at[c].at[pl.ds(s * DSTRIPE, DSTRIPE)],
    )


_deg_kernel = functools.partial(
    pl.kernel,
    out_type=jax.ShapeDtypeStruct((NC, DEG_LEN), jnp.float32),
    mesh=plsc.VectorSubcoreMesh(core_axis_name="c", subcore_axis_name="s"),
    scratch_types=[
        pltpu.VMEM((CPW, CHUNK), jnp.int32),
        pltpu.VMEM((CHUNK,), jnp.float32),
        pltpu.VMEM((DSTRIPE,), jnp.float32),
        pltpu.VMEM_SHARED((DEG_LEN,), jnp.float32),
    ],
)(_deg_body)


G = 5
NB = 8


def _agg_body(lo_hbm, hi_hbm, src_hbm, dst_hbm, out_hbm, idx_s, idx_d, rows, zbuf, acc_sh, gsem, ssem):
    c = lax.axis_index("c")
    s = lax.axis_index("s")
    wid = c * NS + s

    def fill_z(i, _):
        zbuf[i // 4, pl.ds((i % 4) * 16, 16)] = jnp.zeros((16,), jnp.float32)
        return 0

    lax.fori_loop(0, ZROWS * 4, fill_z, 0)

    pltpu.sync_copy(src_hbm.at[wid], idx_s)
    pltpu.sync_copy(dst_hbm.at[wid], idx_d)

    for half, hs_hbm in ((0, lo_hbm), (1, hi_hbm)):
        for k in range(STRIPE // ZROWS):
            pltpu.sync_copy(zbuf, acc_sh.at[pl.ds(s * STRIPE + k * ZROWS, ZROWS)])
        plsc.subcore_barrier()

        def wait_gather():
            pltpu.make_async_copy(hs_hbm.at[idx_s.at[0]], rows.at[0], gsem).wait()

        def wait_scatter():
            pltpu.make_async_copy(rows.at[0], acc_sh.at[idx_d.at[0]], ssem).wait()

        for j in range(G):
            pltpu.async_copy(hs_hbm.at[idx_s.at[j]], rows.at[j % NB], gsem)

        def body(j, _):
            wait_gather()
            @pl.when(j >= NB - G)
            def _():
                wait_scatter()

            pltpu.async_copy(hs_hbm.at[idx_s.at[j + G]], rows.at[(j + G) % NB], gsem)
            pltpu.async_copy(rows.at[j % NB], acc_sh.at[idx_d.at[j]], ssem, add=True)
            return 0

        lax.fori_loop(0, CPW - G, body, 0)
        for j in range(CPW - G, CPW):
            wait_gather()
            wait_scatter()
            pltpu.async_copy(rows.at[j % NB], acc_sh.at[idx_d.at[j]], ssem, add=True)
        for _ in range(NB - G):
            wait_scatter()

        plsc.subcore_barrier()
        pltpu.sync_copy(
            acc_sh.at[pl.ds(s * STRIPE, STRIPE)],
            out_hbm.at[c].at[half].at[pl.ds(s * STRIPE, STRIPE)],
        )
        if half == 0:
            plsc.subcore_barrier()


_agg_kernel = functools.partial(
    pl.kernel,
    out_type=jax.ShapeDtypeStruct((NC, 2, NP, DH), jnp.float32),
    mesh=plsc.VectorSubcoreMesh(core_axis_name="c", subcore_axis_name="s"),
    scratch_types=[
        pltpu.VMEM((CPW, CHUNK), jnp.int32),
        pltpu.VMEM((CPW, CHUNK), jnp.int32),
        pltpu.VMEM((NB, CHUNK, DH), jnp.float32),
        pltpu.VMEM((ZROWS, DH), jnp.float32),
        pltpu.VMEM_SHARED((NP, DH), jnp.float32),
        pltpu.SemaphoreType.DMA,
        pltpu.SemaphoreType.DMA,
    ],
    compiler_params=pltpu.CompilerParams(use_tc_tiling_on_sc=False),
)(_agg_body)


def _dinv(deg_ref):
    return lax.rsqrt(jnp.sum(deg_ref[...], axis=1) + 1.0)


def _split_store(o_lo, o_hi, v):
    o_lo[...] = v[:, :DH]
    o_hi[...] = v[:, DH:]


def _prep_body(x_ref, w_ref, deg_ref, o_lo, o_hi):
    dinv = _dinv(deg_ref)
    h = jnp.dot(x_ref[...], w_ref[...], preferred_element_type=jnp.float32)
    _split_store(o_lo, o_hi, h * dinv[:, None])


def _mid_body(agg_ref, lo_ref, hi_ref, deg_ref, b_ref, w_ref, o_lo, o_hi):
    dinv = _dinv(deg_ref)
    hs = jnp.concatenate([lo_ref[...], hi_ref[...]], axis=1)
    lo = agg_ref[0, 0] + agg_ref[1, 0]
    hi = agg_ref[0, 1] + agg_ref[1, 1]
    p = (jnp.concatenate([lo, hi], axis=1) + hs) * dinv[:, None] + b_ref[...]
    h = jnp.maximum(p, 0.0)
    out = jnp.dot(h, w_ref[...], preferred_element_type=jnp.float32) * dinv[:, None]
    _split_store(o_lo, o_hi, out)


def _final_body(agg_ref, lo_ref, hi_ref, deg_ref, b_ref, wc_ref, bc_ref, o_ref, acc):
    i = pl.program_id(0)
    dinv = _dinv(deg_ref)
    hs = jnp.concatenate([lo_ref[...], hi_ref[...]], axis=1)
    lo = agg_ref[0, 0] + agg_ref[1, 0]
    hi = agg_ref[0, 1] + agg_ref[1, 1]
    p = (jnp.concatenate([lo, hi], axis=1) + hs) * dinv[:, None] + b_ref[...]
    h = jnp.maximum(p, 0.0)
    row = lax.broadcasted_iota(jnp.int32, (BLK, 1), 0) + i * BLK
    h = jnp.where(row < N_NODES, h, 0.0)
    part = jnp.sum(h, axis=0, keepdims=True)

    @pl.when(i == 0)
    def _():
        acc[...] = part

    @pl.when(i > 0)
    def _():
        acc[...] = acc[...] + part

    @pl.when(i == pl.num_programs(0) - 1)
    def _():
        pooled = acc[...] * (1.0 / N_NODES)
        o_ref[...] = (
            jnp.dot(pooled, wc_ref[...], preferred_element_type=jnp.float32)
            + bc_ref[...]
        )


_row_spec = pl.BlockSpec((BLK, D), lambda i: (i, 0))
_half_spec = pl.BlockSpec((BLK, DH), lambda i: (i, 0))
_w_spec = pl.BlockSpec((D, D), lambda i: (0, 0))
_deg_spec = pl.BlockSpec((BLK, NC), lambda i: (i, 0))
_agg_spec = pl.BlockSpec((NC, 2, BLK, DH), lambda i: (0, 0, i, 0))
_b_spec = pl.BlockSpec((D,), lambda i: (0,))

_halves_t = (
    jax.ShapeDtypeStruct((NP, DH), jnp.float32),
    jax.ShapeDtypeStruct((NP, DH), jnp.float32),
)

_prep = pl.pallas_call(
    _prep_body,
    grid=(_GRID,),
    in_specs=[_row_spec, _w_spec, _deg_spec],
    out_specs=(_half_spec, _half_spec),
    out_shape=_halves_t,
)

_mid = pl.pallas_call(
    _mid_body,
    grid=(_GRID,),
    in_specs=[_agg_spec, _half_spec, _half_spec, _deg_spec, _b_spec, _w_spec],
    out_specs=(_half_spec, _half_spec),
    out_shape=_halves_t,
)

_final = pl.pallas_call(
    _final_body,
    grid=(_GRID,),
    in_specs=[
        _agg_spec,
        _half_spec,
        _half_spec,
        _deg_spec,
        _b_spec,
        pl.BlockSpec((D, D_OUT), lambda i: (0, 0)),
        pl.BlockSpec((D_OUT,), lambda i: (0,)),
    ],
    out_specs=pl.BlockSpec((1, D_OUT), lambda i: (0, 0)),
    out_shape=jax.ShapeDtypeStruct((1, D_OUT), jnp.float32),
    scratch_shapes=[pltpu.VMEM((1, D), jnp.float32)],
)


def kernel(x, edge_index, W1, b1, W2, b2, Wc, bc):
    src = edge_index[0].astype(jnp.int32).reshape(NW, CPW, CHUNK)
    dst = edge_index[1].astype(jnp.int32).reshape(NW, CPW, CHUNK)

    xp = jnp.zeros((NP, D), jnp.float32).at[:N_NODES].set(x)

    degp = _deg_kernel(dst)
    deg = jnp.zeros((NP, NC), jnp.float32).at[:N_NODES].set(degp[:, :N_NODES].T)

    hs1_lo, hs1_hi = _prep(xp, W1, deg)
    agg1 = _agg_kernel(hs1_lo, hs1_hi, src, dst)
    hs2_lo, hs2_hi = _mid(agg1, hs1_lo, hs1_hi, deg, b1, W2)
    agg2 = _agg_kernel(hs2_lo, hs2_hi, src, dst)
    out = _final(agg2, hs2_lo, hs2_hi, deg, b2, Wc, bc)
    return out.reshape(D_OUT)
